# speedup vs baseline: 1.0225x; 1.0225x over previous
import jax
import jax.numpy as jnp
from jax.experimental import pallas as pl

_S = 64.0
_MARGIN = 0.4
_ROWS_PER_BLOCK = 16


def _cosface_block(lab_ref, x_ref, o_ref):
    x = x_ref[...]
    lab = lab_ref[...]
    cols = jax.lax.broadcasted_iota(jnp.int32, x.shape, 1)
    mask = cols == lab
    o_ref[...] = x * _S + jnp.where(mask, -_MARGIN * _S, 0.0)


@jax.jit
def kernel(logits, labels):
    B, V = logits.shape
    R = _ROWS_PER_BLOCK
    grid = (B // R,)
    return pl.pallas_call(
        _cosface_block,
        grid=grid,
        in_specs=[
            pl.BlockSpec((R, 1), lambda i: (i, 0)),
            pl.BlockSpec((R, V), lambda i: (i, 0)),
        ],
        out_specs=pl.BlockSpec((R, V), lambda i: (i, 0)),
        out_shape=jax.ShapeDtypeStruct((B, V), logits.dtype),
    )(labels.reshape(B, 1), logits)
